# baseline (speedup 1.0000x reference)
import dataclasses
import functools

import jax
import jax.numpy as jnp
from jax import lax
from jax.experimental import pallas as pl
from jax.experimental.pallas import tpu as pltpu
from jax.experimental.pallas import tpu_sc as plsc

VOXEL_SIZE = 0.02
GRID = 50
NV = GRID ** 3
N = 2_000_000

NC, NS = 2, 16
L = 16
HB = 8192
ACC = NS * HB
BLK = 2048
SB = 496
HALF = SB * BLK
NPAD = NC * HALF
NROWCH = NPAD // 128

PB = 8192
NBLK = NPAD // PB
LAST_IN_BLK = (N + PB - 1) // PB - 1

_cp = pltpu.CompilerParams()
if "needs_layout_passes" in pltpu.CompilerParams.__dataclass_fields__:
    _cp = dataclasses.replace(_cp, needs_layout_passes=False)
if "use_tc_tiling_on_sc" in pltpu.CompilerParams.__dataclass_fields__:
    _cp = dataclasses.replace(_cp, use_tc_tiling_on_sc=False)


def _prep_body(p_ref, c_ref, rows_ref, lin_ref):
    i = pl.program_id(0)
    p = p_ref[...]
    c = c_ref[...]
    vox = jnp.clip(jnp.floor(p * jnp.float32(GRID)), 0.0, jnp.float32(GRID - 1))
    linf = (vox[:, 0:1] * jnp.float32(GRID * GRID)
            + vox[:, 1:2] * jnp.float32(GRID)
            + vox[:, 2:3])
    row = i * PB + lax.broadcasted_iota(jnp.int32, (PB, 1), 0)
    valid = row < N
    lin_ref[...] = jnp.where(valid, linf.astype(jnp.int32), row % NV)
    packed = jnp.concatenate(
        [p, c, jnp.ones((PB, 1), jnp.float32), jnp.zeros((PB, 1), jnp.float32)],
        axis=1)
    rows_ref[...] = jnp.where(valid, packed, 0.0)


_prep = pl.pallas_call(
    _prep_body,
    grid=(NBLK,),
    in_specs=[
        pl.BlockSpec((PB, 3), lambda i: (jnp.minimum(i, LAST_IN_BLK), 0)),
        pl.BlockSpec((PB, 3), lambda i: (jnp.minimum(i, LAST_IN_BLK), 0)),
    ],
    out_specs=[
        pl.BlockSpec((PB, 8), lambda i: (i, 0)),
        pl.BlockSpec((PB, 1), lambda i: (i, 0)),
    ],
    out_shape=[
        jax.ShapeDtypeStruct((NPAD, 8), jnp.float32),
        jax.ShapeDtypeStruct((NPAD, 1), jnp.int32),
    ],
)


_sc_mesh = plsc.VectorSubcoreMesh(core_axis_name="c", subcore_axis_name="s")


@functools.partial(
    pl.kernel,
    out_type=jax.ShapeDtypeStruct((NC, ACC, 8), jnp.float32),
    mesh=_sc_mesh,
    scratch_types=[
        pltpu.VMEM((BLK, 8), jnp.float32),
        pltpu.VMEM((BLK, 8), jnp.float32),
        pltpu.VMEM((16, 128), jnp.int32),
        pltpu.VMEM((16, 128), jnp.int32),
        pltpu.VMEM((HB, 8), jnp.float32),
        pltpu.SemaphoreType.DMA,
        pltpu.SemaphoreType.DMA,
        pltpu.SemaphoreType.DMA,
        pltpu.SemaphoreType.DMA,
    ],
    compiler_params=_cp,
)
def _sc_scatter(rows_hbm, lin_hbm, zeros_hbm, out_hbm,
                rows_a, rows_b, idx_a, idx_b, hist_v,
                sem_ra, sem_rb, sem_ia, sem_ib):
    c = lax.axis_index("c")
    s = lax.axis_index("s")
    pltpu.sync_copy(zeros_hbm, hist_v)
    row_base = c * HALF
    ch_base = c * (HALF // 128)
    base_bin = s * HB
    iota = lax.iota(jnp.int32, L)
    csplat = [jnp.full((L,), col, jnp.int32) for col in range(7)]

    def start(b, rows_v, idx_v, sem_r, sem_i):
        pltpu.async_copy(rows_hbm.at[pl.ds(row_base + b * BLK, BLK)],
                         rows_v, sem_r)
        pltpu.async_copy(lin_hbm.at[pl.ds(ch_base + b * 16, 16)],
                         idx_v, sem_i)

    def wait(b, rows_v, idx_v, sem_r, sem_i):
        pltpu.make_async_copy(rows_hbm.at[pl.ds(row_base + b * BLK, BLK)],
                              rows_v, sem_r).wait()
        pltpu.make_async_copy(lin_hbm.at[pl.ds(ch_base + b * 16, 16)],
                              idx_v, sem_i).wait()

    def compute(rows_v, idx_v):
        @pl.loop(0, 16)
        def _(k):
            for j in range(8):
                v = idx_v[k, pl.ds(j * L, L)]
                local = v - base_bin
                inb = jnp.logical_and(local >= 0, local < HB)
                tid = jnp.where(inb, local, 0)
                rowi = k * 128 + j * L + iota

                @pl.when(jnp.any(inb))
                def _():
                    for col in range(7):
                        vals = plsc.load_gather(rows_v, [rowi, csplat[col]])
                        plsc.addupdate_scatter(
                            hist_v, [tid, csplat[col]], vals, mask=inb)

    start(0, rows_a, idx_a, sem_ra, sem_ia)

    @pl.loop(0, SB // 2)
    def _(h):
        b0 = 2 * h
        b1 = b0 + 1
        wait(b0, rows_a, idx_a, sem_ra, sem_ia)
        start(b1, rows_b, idx_b, sem_rb, sem_ib)
        compute(rows_a, idx_a)
        wait(b1, rows_b, idx_b, sem_rb, sem_ib)

        @pl.when(b1 + 1 < SB)
        def _():
            start(b1 + 1, rows_a, idx_a, sem_ra, sem_ia)

        compute(rows_b, idx_b)

    pltpu.sync_copy(hist_v, out_hbm.at[c].at[pl.ds(s * HB, HB)])


FB = 5000
NFB = NV // FB


def _final_body(p_ref, avgp_ref, avgc_ref, cnt_ref):
    ssum = p_ref[0] + p_ref[1]
    cnt = ssum[:, 6:7]
    denom = jnp.maximum(cnt, 1.0)
    avgp_ref[...] = ssum[:, 0:3] / denom
    avgc_ref[...] = ssum[:, 3:6] / denom
    cnt_ref[...] = cnt


_final = pl.pallas_call(
    _final_body,
    grid=(NFB,),
    in_specs=[pl.BlockSpec((NC, FB, 8), lambda i: (0, i, 0))],
    out_specs=[
        pl.BlockSpec((FB, 3), lambda i: (i, 0)),
        pl.BlockSpec((FB, 3), lambda i: (i, 0)),
        pl.BlockSpec((FB, 1), lambda i: (i, 0)),
    ],
    out_shape=[
        jax.ShapeDtypeStruct((NV, 3), jnp.float32),
        jax.ShapeDtypeStruct((NV, 3), jnp.float32),
        jax.ShapeDtypeStruct((NV, 1), jnp.float32),
    ],
)


def kernel(points, colors):
    rows8, lin = _prep(points, colors)
    lin2 = lin.reshape(NROWCH, 128)
    zeros = jnp.zeros((HB, 8), jnp.float32)
    partials = _sc_scatter(rows8, lin2, zeros)
    avg_points, avg_colors, counts = _final(partials)
    return avg_points, avg_colors, counts.reshape(NV)


# speedup vs baseline: 1.1728x; 1.1728x over previous
import dataclasses
import functools

import jax
import jax.numpy as jnp
from jax import lax
from jax.experimental import pallas as pl
from jax.experimental.pallas import tpu as pltpu
from jax.experimental.pallas import tpu_sc as plsc

VOXEL_SIZE = 0.02
GRID = 50
NV = GRID ** 3
N = 2_000_000

NC, NS = 2, 16
L = 16
HB = 8192
ACC = NS * HB
BLK = 2048
SB = 496
HALF = SB * BLK
NPAD = NC * HALF
NROWCH = NPAD // 128

PB = 8192
NBLK = NPAD // PB
LAST_IN_BLK = (N + PB - 1) // PB - 1

_cp = pltpu.CompilerParams()
if "needs_layout_passes" in pltpu.CompilerParams.__dataclass_fields__:
    _cp = dataclasses.replace(_cp, needs_layout_passes=False)
if "use_tc_tiling_on_sc" in pltpu.CompilerParams.__dataclass_fields__:
    _cp = dataclasses.replace(_cp, use_tc_tiling_on_sc=False)


def _prep_body(p_ref, c_ref, rows_ref, lin_ref):
    i = pl.program_id(0)
    p = p_ref[...]
    c = c_ref[...]
    vox = jnp.clip(jnp.floor(p * jnp.float32(GRID)), 0.0, jnp.float32(GRID - 1))
    linf = (vox[:, 0:1] * jnp.float32(GRID * GRID)
            + vox[:, 1:2] * jnp.float32(GRID)
            + vox[:, 2:3])
    row = i * PB + lax.broadcasted_iota(jnp.int32, (PB, 1), 0)
    valid = row < N
    lin_ref[...] = jnp.where(valid, linf.astype(jnp.int32), row % NV)
    packed = jnp.concatenate(
        [p, c, jnp.ones((PB, 1), jnp.float32), jnp.zeros((PB, 1), jnp.float32)],
        axis=1)
    rows_ref[...] = jnp.where(valid, packed, 0.0)


_prep = pl.pallas_call(
    _prep_body,
    grid=(NBLK,),
    in_specs=[
        pl.BlockSpec((PB, 3), lambda i: (jnp.minimum(i, LAST_IN_BLK), 0)),
        pl.BlockSpec((PB, 3), lambda i: (jnp.minimum(i, LAST_IN_BLK), 0)),
    ],
    out_specs=[
        pl.BlockSpec((PB, 8), lambda i: (i, 0)),
        pl.BlockSpec((PB, 1), lambda i: (i, 0)),
    ],
    out_shape=[
        jax.ShapeDtypeStruct((NPAD, 8), jnp.float32),
        jax.ShapeDtypeStruct((NPAD, 1), jnp.int32),
    ],
)


_sc_mesh = plsc.VectorSubcoreMesh(core_axis_name="c", subcore_axis_name="s")


@functools.partial(
    pl.kernel,
    out_type=jax.ShapeDtypeStruct((NC, ACC, 8), jnp.float32),
    mesh=_sc_mesh,
    scratch_types=[
        pltpu.VMEM((BLK, 8), jnp.float32),
        pltpu.VMEM((BLK, 8), jnp.float32),
        pltpu.VMEM((16, 128), jnp.int32),
        pltpu.VMEM((16, 128), jnp.int32),
        pltpu.VMEM((HB, 8), jnp.float32),
        pltpu.SemaphoreType.DMA,
        pltpu.SemaphoreType.DMA,
        pltpu.SemaphoreType.DMA,
        pltpu.SemaphoreType.DMA,
    ],
    compiler_params=_cp,
)
def _sc_scatter(rows_hbm, lin_hbm, zeros_hbm, out_hbm,
                rows_a, rows_b, idx_a, idx_b, hist_v,
                sem_ra, sem_rb, sem_ia, sem_ib):
    c = lax.axis_index("c")
    s = lax.axis_index("s")
    pltpu.sync_copy(zeros_hbm, hist_v)
    row_base = c * HALF
    ch_base = c * (HALF // 128)
    base_bin = s * HB
    iota = lax.iota(jnp.int32, L)
    csplat = [jnp.full((L,), col, jnp.int32) for col in range(7)]

    def start(b, rows_v, idx_v, sem_r, sem_i):
        pltpu.async_copy(rows_hbm.at[pl.ds(row_base + b * BLK, BLK)],
                         rows_v, sem_r)
        pltpu.async_copy(lin_hbm.at[pl.ds(ch_base + b * 16, 16)],
                         idx_v, sem_i)

    def wait(b, rows_v, idx_v, sem_r, sem_i):
        pltpu.make_async_copy(rows_hbm.at[pl.ds(row_base + b * BLK, BLK)],
                              rows_v, sem_r).wait()
        pltpu.make_async_copy(lin_hbm.at[pl.ds(ch_base + b * 16, 16)],
                              idx_v, sem_i).wait()

    def compute(rows_v, idx_v):
        @pl.loop(0, 16)
        def _(k):
            for j in range(8):
                v = idx_v[k, pl.ds(j * L, L)]
                local = v - base_bin
                inb = jnp.logical_and(local >= 0, local < HB)
                tid = jnp.where(inb, local, 0)
                rowi = k * 128 + j * L + iota
                for col in range(7):
                    vals = plsc.load_gather(rows_v, [rowi, csplat[col]])
                    plsc.addupdate_scatter(
                        hist_v, [tid, csplat[col]], vals, mask=inb)

    start(0, rows_a, idx_a, sem_ra, sem_ia)

    @pl.loop(0, SB // 2)
    def _(h):
        b0 = 2 * h
        b1 = b0 + 1
        wait(b0, rows_a, idx_a, sem_ra, sem_ia)
        start(b1, rows_b, idx_b, sem_rb, sem_ib)
        compute(rows_a, idx_a)
        wait(b1, rows_b, idx_b, sem_rb, sem_ib)

        @pl.when(b1 + 1 < SB)
        def _():
            start(b1 + 1, rows_a, idx_a, sem_ra, sem_ia)

        compute(rows_b, idx_b)

    pltpu.sync_copy(hist_v, out_hbm.at[c].at[pl.ds(s * HB, HB)])


FB = 5000
NFB = NV // FB


def _final_body(p_ref, avgp_ref, avgc_ref, cnt_ref):
    ssum = p_ref[0] + p_ref[1]
    cnt = ssum[:, 6:7]
    denom = jnp.maximum(cnt, 1.0)
    avgp_ref[...] = ssum[:, 0:3] / denom
    avgc_ref[...] = ssum[:, 3:6] / denom
    cnt_ref[...] = cnt


_final = pl.pallas_call(
    _final_body,
    grid=(NFB,),
    in_specs=[pl.BlockSpec((NC, FB, 8), lambda i: (0, i, 0))],
    out_specs=[
        pl.BlockSpec((FB, 3), lambda i: (i, 0)),
        pl.BlockSpec((FB, 3), lambda i: (i, 0)),
        pl.BlockSpec((FB, 1), lambda i: (i, 0)),
    ],
    out_shape=[
        jax.ShapeDtypeStruct((NV, 3), jnp.float32),
        jax.ShapeDtypeStruct((NV, 3), jnp.float32),
        jax.ShapeDtypeStruct((NV, 1), jnp.float32),
    ],
)


def kernel(points, colors):
    rows8, lin = _prep(points, colors)
    lin2 = lin.reshape(NROWCH, 128)
    zeros = jnp.zeros((HB, 8), jnp.float32)
    partials = _sc_scatter(rows8, lin2, zeros)
    avg_points, avg_colors, counts = _final(partials)
    return avg_points, avg_colors, counts.reshape(NV)
